# Initial kernel scaffold; baseline (speedup 1.0000x reference)
import jax
import jax.numpy as jnp
from jax.experimental import pallas as pl


def kernel(embed, learn_lut):
    raise NotImplementedError("write your pallas kernel here")


# speedup vs baseline: 2.8239x; 2.8239x over previous
import jax
import jax.numpy as jnp
from jax.experimental import pallas as pl


def _posenc_add_kernel(e_ref, l_ref, o_ref):
    o_ref[...] = e_ref[...] + l_ref[...]


def kernel(embed, learn_lut):
    B, S, D = embed.shape
    TS = 512
    grid = (S // TS, B)
    return pl.pallas_call(
        _posenc_add_kernel,
        grid=grid,
        in_specs=[
            pl.BlockSpec((1, TS, D), lambda i, b: (b, i, 0)),
            pl.BlockSpec((TS, D), lambda i, b: (i, 0)),
        ],
        out_specs=pl.BlockSpec((1, TS, D), lambda i, b: (b, i, 0)),
        out_shape=jax.ShapeDtypeStruct((B, S, D), embed.dtype),
    )(embed, learn_lut[:S])


# speedup vs baseline: 3.1275x; 1.1075x over previous
import jax
import jax.numpy as jnp
from jax.experimental import pallas as pl


def _posenc_add_kernel(e_ref, l_ref, o_ref):
    o_ref[...] = e_ref[...] + l_ref[...]


def kernel(embed, learn_lut):
    B, S, D = embed.shape
    TS = 1024
    grid = (S // TS, B)
    return pl.pallas_call(
        _posenc_add_kernel,
        grid=grid,
        in_specs=[
            pl.BlockSpec((1, TS, D), lambda i, b: (b, i, 0)),
            pl.BlockSpec((TS, D), lambda i, b: (i, 0)),
        ],
        out_specs=pl.BlockSpec((1, TS, D), lambda i, b: (b, i, 0)),
        out_shape=jax.ShapeDtypeStruct((B, S, D), embed.dtype),
    )(embed, learn_lut[:S])


# speedup vs baseline: 3.3166x; 1.0605x over previous
import jax
import jax.numpy as jnp
from jax.experimental import pallas as pl


def _posenc_add_kernel(e_ref, l_ref, o_ref):
    o_ref[...] = e_ref[...] + l_ref[...]


def kernel(embed, learn_lut):
    B, S, D = embed.shape
    TS = 2048
    grid = (S // TS, B)
    return pl.pallas_call(
        _posenc_add_kernel,
        grid=grid,
        in_specs=[
            pl.BlockSpec((1, TS, D), lambda i, b: (b, i, 0)),
            pl.BlockSpec((TS, D), lambda i, b: (i, 0)),
        ],
        out_specs=pl.BlockSpec((1, TS, D), lambda i, b: (b, i, 0)),
        out_shape=jax.ShapeDtypeStruct((B, S, D), embed.dtype),
    )(embed, learn_lut[:S])
